# baseline (speedup 1.0000x reference)
import functools
import math

import jax
import jax.numpy as jnp
from jax import lax
from jax.experimental import pallas as pl
from jax.experimental.pallas import tpu as pltpu
from jax.experimental.pallas import tpu_sc as plsc

D = 64
SCALE = math.sqrt(D)
NC = 2
NS = 16
NW = NC * NS
VOCAB = 1000000
CH = 128
NBUF = 3

_mesh = lambda: plsc.VectorSubcoreMesh(core_axis_name="c", subcore_axis_name="s")
_params = lambda: pltpu.CompilerParams(
    use_tc_tiling_on_sc=True, needs_layout_passes=False)


def _gather_kernel(B):
    per_w = B // NW
    n_ch = per_w // CH

    @functools.partial(
        pl.kernel,
        mesh=_mesh(),
        compiler_params=_params(),
        out_type=jax.ShapeDtypeStruct((B, 2 * D), jnp.float32),
        scratch_types=(
            [pltpu.VMEM((n_ch, CH), jnp.int32)]
            + [pltpu.VMEM((CH, 2 * D), jnp.float32)] * (NBUF + 2)
            + [pltpu.SemaphoreType.DMA] * (NBUF + 2)
        ),
    )
    def gk(xF, tabD, out2, *rest):
        idxv = rest[0]
        gs = rest[1:1 + NBUF]
        ss = rest[1 + NBUF:3 + NBUF]
        sg = rest[3 + NBUF:3 + 2 * NBUF]
        so = rest[3 + 2 * NBUF:5 + 2 * NBUF]
        wid = lax.axis_index("s") * NC + lax.axis_index("c")
        base = wid * per_w

        pltpu.sync_copy(xF.at[pl.ds(wid * n_ch, n_ch)], idxv)

        def start_gather(c, s):
            pltpu.async_copy(tabD.at[idxv.at[c]], gs[s], sg[s])

        def wait_gather(s):
            pltpu.make_async_copy(tabD.at[idxv.at[0]], gs[s], sg[s]).wait()

        def start_out(c, s):
            pltpu.async_copy(
                ss[s], out2.at[pl.ds(base + c * CH, CH)], so[s])

        def wait_out(s):
            pltpu.make_async_copy(
                ss[s], out2.at[pl.ds(base, CH)], so[s]).wait()

        def scale(gsl, osl):
            g = gs[gsl]
            o = ss[osl]

            @plsc.parallel_loop(0, CH, unroll=4)
            def _(p):
                for cb in range(D // 16):
                    sl = pl.ds(cb * 16, 16)
                    o[p, sl] = g[p, sl] * SCALE

        for k in range(NBUF):
            start_gather(k, k)

        def body(t, carry):
            gslot = lax.rem(t, NBUF)
            oslot = lax.rem(t, 2)

            def stage(gsl, osl):
                wait_gather(gsl)

                @pl.when(t >= 2)
                def _():
                    wait_out(osl)
                scale(gsl, osl)
                start_out(t, osl)

                @pl.when(t + NBUF < n_ch)
                def _():
                    start_gather(t + NBUF, gsl)

            for gsl in range(NBUF):
                @pl.when(gslot == gsl)
                def _(gsl=gsl):
                    for osl in range(2):
                        @pl.when(oslot == osl)
                        def _(osl=osl):
                            stage(gsl, osl)
            return carry

        lax.fori_loop(0, n_ch, body, 0)
        wait_out(0)
        wait_out(1)

    return gk


def kernel(x, table):
    S0, S1 = x.shape
    B = S0 * S1
    xF = x.reshape(B // CH, CH).astype(jnp.int32)
    tabD = jnp.pad(table, ((0, 0), (0, D)))
    out2 = _gather_kernel(B)(xF, tabD)
    return out2[:, :D].reshape(S0, S1, D)
